# baseline (speedup 1.0000x reference)
import functools
import numpy as np
import jax
from jax import lax
import jax.numpy as jnp
from jax.experimental import pallas as pl
from jax.experimental.pallas import tpu as pltpu
from jax.experimental.pallas import tpu_sc as plsc

_SLOW_FRAMES = 8
_PH = 32
_K = 14
_LEAD = 7


def _slow_indices(t):
    return tuple(int(v) for v in np.linspace(0, t - 1, _SLOW_FRAMES).astype(np.int32))


def kernel(frames):
    b, t, c, h, w = frames.shape
    n_slow = _SLOW_FRAMES
    assert _slow_indices(t) == tuple((j * (t - 1)) // (n_slow - 1) for j in range(n_slow))
    mesh = plsc.VectorSubcoreMesh(core_axis_name="c", subcore_axis_name="s")
    n_workers = 32
    chunks = b * n_slow
    per_w = chunks // n_workers
    pp_h = h // _PH
    ppc = c * pp_h
    n_pieces = per_w * ppc

    @functools.partial(
        pl.kernel,
        mesh=mesh,
        out_type=jax.ShapeDtypeStruct((b, n_slow, c, h, w), frames.dtype),
        scratch_types=(
            [pltpu.VMEM((_PH, w), frames.dtype)] * _K
            + [pltpu.SemaphoreType.DMA] * (2 * _K)
        ),
    )
    def sc_gather(frames_hbm, slow_hbm, *scratch):
        bufs = scratch[:_K]
        in_sems = scratch[_K : 2 * _K]
        out_sems = scratch[2 * _K : 3 * _K]
        cid = lax.axis_index("c")
        sid = lax.axis_index("s")
        wid = sid * 2 + cid

        def coords(p):
            r = wid * per_w + p // ppc
            q = p % ppc
            ci, hp = q // pp_h, q % pp_h
            bi = r // n_slow
            j = r % n_slow
            ti = (j * (t - 1)) // (n_slow - 1)
            return bi, j, ti, ci, hp * _PH

        ins, outs = {}, {}
        for step in range(n_pieces + _LEAD):
            if step < n_pieces:
                k = step % _K
                bi, j, ti, ci, row0 = coords(step)
                if step >= _K:
                    outs[step - _K].wait()
                cp = pltpu.make_async_copy(
                    frames_hbm.at[bi, ti, ci, pl.ds(row0, _PH)], bufs[k], in_sems[k]
                )
                cp.start()
                ins[step] = cp
            r = step - _LEAD
            if r >= 0:
                k = r % _K
                bi, j, ti, ci, row0 = coords(r)
                ins[r].wait()
                cp = pltpu.make_async_copy(
                    bufs[k], slow_hbm.at[bi, j, ci, pl.ds(row0, _PH)], out_sems[k]
                )
                cp.start()
                outs[r] = cp
        for r in range(max(0, n_pieces - _K), n_pieces):
            outs[r].wait()

    slow = sc_gather(frames)
    return (slow, frames)


# speedup vs baseline: 1.0123x; 1.0123x over previous
import functools
import numpy as np
import jax
from jax import lax
import jax.numpy as jnp
from jax.experimental import pallas as pl
from jax.experimental.pallas import tpu as pltpu
from jax.experimental.pallas import tpu_sc as plsc

_SLOW_FRAMES = 8
_PH = 64
_K = 7
_LEAD = 3


def _slow_indices(t):
    return tuple(int(v) for v in np.linspace(0, t - 1, _SLOW_FRAMES).astype(np.int32))


def kernel(frames):
    b, t, c, h, w = frames.shape
    n_slow = _SLOW_FRAMES
    assert _slow_indices(t) == tuple((j * (t - 1)) // (n_slow - 1) for j in range(n_slow))
    mesh = plsc.VectorSubcoreMesh(core_axis_name="c", subcore_axis_name="s")
    n_workers = 32
    chunks = b * n_slow
    per_w = chunks // n_workers
    pp_h = h // _PH
    ppc = c * pp_h
    n_pieces = per_w * ppc

    @functools.partial(
        pl.kernel,
        mesh=mesh,
        out_type=jax.ShapeDtypeStruct((b, n_slow, c, h, w), frames.dtype),
        scratch_types=(
            [pltpu.VMEM((_PH, w), frames.dtype)] * _K
            + [pltpu.SemaphoreType.DMA] * (2 * _K)
        ),
    )
    def sc_gather(frames_hbm, slow_hbm, *scratch):
        bufs = scratch[:_K]
        in_sems = scratch[_K : 2 * _K]
        out_sems = scratch[2 * _K : 3 * _K]
        cid = lax.axis_index("c")
        sid = lax.axis_index("s")
        wid = sid * 2 + cid

        def coords(p):
            r = wid * per_w + p // ppc
            q = p % ppc
            ci, hp = q // pp_h, q % pp_h
            bi = r // n_slow
            j = r % n_slow
            ti = (j * (t - 1)) // (n_slow - 1)
            return bi, j, ti, ci, hp * _PH

        ins, outs = {}, {}
        for step in range(n_pieces + _LEAD):
            if step < n_pieces:
                k = step % _K
                bi, j, ti, ci, row0 = coords(step)
                if step >= _K:
                    outs[step - _K].wait()
                cp = pltpu.make_async_copy(
                    frames_hbm.at[bi, ti, ci, pl.ds(row0, _PH)], bufs[k], in_sems[k]
                )
                cp.start()
                ins[step] = cp
            r = step - _LEAD
            if r >= 0:
                k = r % _K
                bi, j, ti, ci, row0 = coords(r)
                ins[r].wait()
                cp = pltpu.make_async_copy(
                    bufs[k], slow_hbm.at[bi, j, ci, pl.ds(row0, _PH)], out_sems[k]
                )
                cp.start()
                outs[r] = cp
        for r in range(max(0, n_pieces - _K), n_pieces):
            outs[r].wait()

    slow = sc_gather(frames)
    return (slow, frames)
